# Initial kernel scaffold; baseline (speedup 1.0000x reference)
import jax
import jax.numpy as jnp
from jax.experimental import pallas as pl


def kernel(x, edge_index, X_param, W1_l, b1_l, W1_r, W2_l, b2_l, W2_r):
    raise NotImplementedError("write your pallas kernel here")


# speedup vs baseline: 1.6977x; 1.6977x over previous
import functools

import jax
import jax.numpy as jnp
from jax import lax
from jax.experimental import pallas as pl
from jax.experimental.pallas import tpu as pltpu
from jax.experimental.pallas import tpu_sc as plsc

_NODES = 10000
_EDGES = 160000
_D = 256

_NW = 32
_WIN = 320
_NPAD = _NW * _WIN
_TRASH = _WIN
_AROWS = _WIN + 8
_CW = 16

_CF = 4000
_NFC = _EDGES // _CF
_C = 64
_EMAX = 160768

_mesh = plsc.VectorSubcoreMesh(core_axis_name="c", subcore_axis_name="s")
_sc_params = pltpu.CompilerParams(needs_layout_passes=False)


@functools.partial(
    pl.kernel,
    out_type=(
        jax.ShapeDtypeStruct((_NW * _EMAX,), jnp.int32),
        jax.ShapeDtypeStruct((_NW * _EMAX,), jnp.int32),
        jax.ShapeDtypeStruct((_NW * _CW,), jnp.int32),
    ),
    mesh=_mesh,
    compiler_params=_sc_params,
    scratch_types=[
        pltpu.VMEM((_CF,), jnp.int32),
        pltpu.VMEM((_CF,), jnp.int32),
        pltpu.VMEM((_CF + 16,), jnp.int32),
        pltpu.VMEM((_CF + 16,), jnp.int32),
        pltpu.VMEM((_CW,), jnp.int32),
    ],
)
def _filter(src_hbm, dst_hbm, srcl_hbm, locl_hbm, nch_hbm,
            fsrc_v, fdst_v, csrc_v, cloc_v, nch_v):
    c = lax.axis_index("c")
    s = lax.axis_index("s")
    w = c * 16 + s
    base = w * _WIN
    lbase = w * _EMAX

    zero16 = jnp.zeros((16,), jnp.int32)
    trash16 = jnp.full((16,), _TRASH, jnp.int32)

    def _scan_chunk(i, out_off):
        pltpu.sync_copy(src_hbm.at[pl.ds(i * _CF, _CF)], fsrc_v)
        pltpu.sync_copy(dst_hbm.at[pl.ds(i * _CF, _CF)], fdst_v)

        def _vec(j, n):
            o = j * 16
            d = fdst_v[pl.ds(o, 16)]
            sv = fsrc_v[pl.ds(o, 16)]
            loc = d - base
            ok = (loc >= 0) & (loc < _WIN)
            plsc.store_compressed(csrc_v.at[pl.ds(n, 16)], sv, mask=ok)
            plsc.store_compressed(cloc_v.at[pl.ds(n, 16)], loc, mask=ok)
            return n + plsc.all_reduce_population_count(ok)[0]

        n = lax.fori_loop(0, _CF // 16, _vec, jnp.int32(0))
        csrc_v[pl.ds(n, 16)] = zero16
        cloc_v[pl.ds(n, 16)] = trash16
        n_r = ((n + 15) // 16) * 16
        dst_off = pl.multiple_of(lbase + out_off, 16)
        pltpu.sync_copy(csrc_v, srcl_hbm.at[pl.ds(dst_off, _CF + 16)])
        pltpu.sync_copy(cloc_v, locl_hbm.at[pl.ds(dst_off, _CF + 16)])
        return out_off + n_r

    n_tot = lax.fori_loop(0, _NFC, _scan_chunk, jnp.int32(0))

    for k in range(5):
        csrc_v[pl.ds(k * 16, 16)] = zero16
        cloc_v[pl.ds(k * 16, 16)] = trash16
    tail_off = pl.multiple_of(lbase + n_tot, 16)
    pltpu.sync_copy(csrc_v.at[pl.ds(0, 80)], srcl_hbm.at[pl.ds(tail_off, 80)])
    pltpu.sync_copy(cloc_v.at[pl.ds(0, 80)], locl_hbm.at[pl.ds(tail_off, 80)])

    nch = (n_tot + _C - 1) // _C
    nch_v[...] = jnp.full((_CW,), 1, jnp.int32) * nch
    pltpu.sync_copy(nch_v, nch_hbm.at[pl.ds(w * _CW, _CW)])


@functools.partial(
    pl.kernel,
    out_type=(
        jax.ShapeDtypeStruct((_NPAD, _D), jnp.float32),
        jax.ShapeDtypeStruct((_NPAD * _CW,), jnp.float32),
    ),
    mesh=_mesh,
    compiler_params=_sc_params,
    scratch_types=[
        pltpu.VMEM((_C,), jnp.int32),
        pltpu.VMEM((_C + 16,), jnp.int32),
        pltpu.VMEM((_C, _D), jnp.float32),
        pltpu.VMEM((_AROWS, _D), jnp.float32),
        pltpu.VMEM((_AROWS * _CW,), jnp.float32),
        pltpu.VMEM((_CW,), jnp.int32),
        pltpu.SemaphoreType.DMA,
    ],
)
def _segsum(z_hbm, srcl_hbm, locl_hbm, nch_hbm, out_hbm, cnt_hbm,
            src_v, loc_v, rows_v, acc_v, cnt_v, nch_v, sem):
    c = lax.axis_index("c")
    s = lax.axis_index("s")
    w = c * 16 + s
    base = w * _WIN
    lbase = w * _EMAX

    zero16 = jnp.zeros((16,), jnp.float32)
    one16 = jnp.ones((16,), jnp.float32)

    def _zacc(r, carry):
        for k in range(_D // 16):
            acc_v[r, pl.ds(k * 16, 16)] = zero16
        cnt_v[pl.ds(r * _CW, _CW)] = zero16
        return carry

    lax.fori_loop(0, _AROWS, _zacc, 0)

    pltpu.sync_copy(nch_hbm.at[pl.ds(w * _CW, _CW)], nch_v)
    nch = nch_v[...][0]

    def _chunk(i, carry):
        off = pl.multiple_of(lbase + i * _C, 16)
        pltpu.sync_copy(srcl_hbm.at[pl.ds(off, _C)], src_v)
        pltpu.sync_copy(locl_hbm.at[pl.ds(off, _C)], loc_v.at[pl.ds(0, _C)])
        pltpu.async_copy(z_hbm.at[src_v], rows_v, sem).wait()

        def _edge(j, carry2):
            loc = loc_v[pl.ds(j, 16)][0]
            plsc.addupdate(cnt_v.at[pl.ds(loc * _CW, _CW)], one16)
            for k in range(_D // 16):
                plsc.addupdate(acc_v.at[loc, pl.ds(k * 16, 16)],
                               rows_v[j, pl.ds(k * 16, 16)])
            return carry2

        lax.fori_loop(0, _C, _edge, 0)
        return carry

    lax.fori_loop(0, nch, _chunk, 0)

    pltpu.sync_copy(acc_v.at[pl.ds(0, _WIN)], out_hbm.at[pl.ds(base, _WIN)])
    pltpu.sync_copy(cnt_v.at[pl.ds(0, _WIN * _CW)],
                    cnt_hbm.at[pl.ds(base * _CW, _WIN * _CW)])


_BR = 1024


def _fused_body(s_ref, c_ref, x_ref, wl_ref, wr_ref, b_ref, o_ref, *, elu):
    cnt = jnp.maximum(c_ref[...][:, 0:1], 1.0)
    mean = s_ref[...] / cnt
    acc = lax.dot_general(mean, wl_ref[...], (((1,), (1,)), ((), ())),
                          precision=lax.Precision.HIGHEST,
                          preferred_element_type=jnp.float32)
    acc = acc + lax.dot_general(x_ref[...], wr_ref[...], (((1,), (1,)), ((), ())),
                                precision=lax.Precision.HIGHEST,
                                preferred_element_type=jnp.float32)
    acc = acc + b_ref[...]
    if elu:
        acc = jnp.where(acc > 0.0, acc, jnp.exp(jnp.minimum(acc, 0.0)) - 1.0)
    o_ref[...] = acc


def _fused(ssum, cnt, x, w_l, w_r, b, elu):
    return pl.pallas_call(
        functools.partial(_fused_body, elu=elu),
        grid=(_NPAD // _BR,),
        in_specs=[
            pl.BlockSpec((_BR, _D), lambda i: (i, 0)),
            pl.BlockSpec((_BR, _CW), lambda i: (i, 0)),
            pl.BlockSpec((_BR, _D), lambda i: (i, 0)),
            pl.BlockSpec((_D, _D), lambda i: (0, 0)),
            pl.BlockSpec((_D, _D), lambda i: (0, 0)),
            pl.BlockSpec((1, _D), lambda i: (0, 0)),
        ],
        out_specs=pl.BlockSpec((_BR, _D), lambda i: (i, 0)),
        out_shape=jax.ShapeDtypeStruct((_NPAD, _D), jnp.float32),
    )(ssum, cnt, x, w_l, w_r, b)


def kernel(x, edge_index, X_param, W1_l, b1_l, W1_r, W2_l, b2_l, W2_r):
    del x
    src = edge_index[0].astype(jnp.int32)
    dst = edge_index[1].astype(jnp.int32)
    xp = jnp.pad(X_param, ((0, _NPAD - _NODES), (0, 0)))
    srcl, locl, nch = _filter(src, dst)
    s1, cnt = _segsum(xp, srcl, locl, nch)
    cnt = cnt.reshape(_NPAD, _CW)
    h = _fused(s1, cnt, xp, W1_l, W1_r, b1_l.reshape(1, _D), True)
    s2, _ = _segsum(h, srcl, locl, nch)
    out = _fused(s2, cnt, h, W2_l, W2_r, b2_l.reshape(1, _D), False)
    return out[:_NODES]


# speedup vs baseline: 4.0027x; 2.3577x over previous
import functools

import jax
import jax.numpy as jnp
from jax import lax
from jax.experimental import pallas as pl
from jax.experimental.pallas import tpu as pltpu
from jax.experimental.pallas import tpu_sc as plsc

_NODES = 10000
_EDGES = 160000
_D = 256

_NW = 32
_WIN = 320
_NPAD = _NW * _WIN
_TRASH = _WIN
_AROWS = _WIN + 8
_CW = 16

_CF = 4000
_NFC = _EDGES // _CF
_C = 64
_EMAX = 160768

_mesh = plsc.VectorSubcoreMesh(core_axis_name="c", subcore_axis_name="s")
_sc_params = pltpu.CompilerParams(needs_layout_passes=False)


@functools.partial(
    pl.kernel,
    out_type=(
        jax.ShapeDtypeStruct((_NW * _EMAX,), jnp.int32),
        jax.ShapeDtypeStruct((_NW * _EMAX,), jnp.int32),
        jax.ShapeDtypeStruct((_NW * _CW,), jnp.int32),
    ),
    mesh=_mesh,
    compiler_params=_sc_params,
    scratch_types=[
        pltpu.VMEM((_CF,), jnp.int32),
        pltpu.VMEM((_CF,), jnp.int32),
        pltpu.VMEM((_CF + 16,), jnp.int32),
        pltpu.VMEM((_CF + 16,), jnp.int32),
        pltpu.VMEM((_CW,), jnp.int32),
    ],
)
def _filter(src_hbm, dst_hbm, srcl_hbm, locl_hbm, nch_hbm,
            fsrc_v, fdst_v, csrc_v, cloc_v, nch_v):
    c = lax.axis_index("c")
    s = lax.axis_index("s")
    w = c * 16 + s
    base = w * _WIN
    lbase = w * _EMAX

    padsrc16 = jnp.full((16,), 1, jnp.int32) * base
    trash16 = jnp.full((16,), _TRASH, jnp.int32)

    def _scan_chunk(i, out_off):
        pltpu.sync_copy(src_hbm.at[pl.ds(i * _CF, _CF)], fsrc_v)
        pltpu.sync_copy(dst_hbm.at[pl.ds(i * _CF, _CF)], fdst_v)

        def _vec(j, n):
            o = j * 16
            d = fdst_v[pl.ds(o, 16)]
            sv = fsrc_v[pl.ds(o, 16)]
            loc = d - base
            ok = (loc >= 0) & (loc < _WIN)
            plsc.store_compressed(csrc_v.at[pl.ds(n, 16)], sv, mask=ok)
            plsc.store_compressed(cloc_v.at[pl.ds(n, 16)], loc, mask=ok)
            return n + plsc.all_reduce_population_count(ok)[0]

        n = lax.fori_loop(0, _CF // 16, _vec, jnp.int32(0))
        csrc_v[pl.ds(n, 16)] = padsrc16
        cloc_v[pl.ds(n, 16)] = trash16
        n_r = ((n + 15) // 16) * 16
        dst_off = pl.multiple_of(lbase + out_off, 16)
        pltpu.sync_copy(csrc_v, srcl_hbm.at[pl.ds(dst_off, _CF + 16)])
        pltpu.sync_copy(cloc_v, locl_hbm.at[pl.ds(dst_off, _CF + 16)])
        return out_off + n_r

    n_tot = lax.fori_loop(0, _NFC, _scan_chunk, jnp.int32(0))

    for k in range(5):
        csrc_v[pl.ds(k * 16, 16)] = padsrc16
        cloc_v[pl.ds(k * 16, 16)] = trash16
    tail_off = pl.multiple_of(lbase + n_tot, 16)
    pltpu.sync_copy(csrc_v.at[pl.ds(0, 80)], srcl_hbm.at[pl.ds(tail_off, 80)])
    pltpu.sync_copy(cloc_v.at[pl.ds(0, 80)], locl_hbm.at[pl.ds(tail_off, 80)])

    nch = (n_tot + _C - 1) // _C
    nch_v[...] = jnp.full((_CW,), 1, jnp.int32) * nch
    pltpu.sync_copy(nch_v, nch_hbm.at[pl.ds(w * _CW, _CW)])


def _make_segsum(with_counts):
    out_type = [jax.ShapeDtypeStruct((_NPAD, _D), jnp.float32)]
    scratch = [
        pltpu.VMEM((_C,), jnp.int32),
        pltpu.VMEM((_C,), jnp.int32),
        pltpu.VMEM((_C + 16,), jnp.int32),
        pltpu.VMEM((_C + 16,), jnp.int32),
        pltpu.VMEM((_C, _D), jnp.float32),
        pltpu.VMEM((_C, _D), jnp.float32),
        pltpu.VMEM((_AROWS, _D), jnp.float32),
        pltpu.VMEM((_CW,), jnp.int32),
        pltpu.SemaphoreType.DMA,
        pltpu.SemaphoreType.DMA,
        pltpu.SemaphoreType.DMA,
        pltpu.SemaphoreType.DMA,
    ]
    if with_counts:
        out_type.append(jax.ShapeDtypeStruct((_NPAD * _CW,), jnp.float32))
        scratch.append(pltpu.VMEM((_AROWS * _CW,), jnp.float32))

    @functools.partial(
        pl.kernel,
        out_type=tuple(out_type) if with_counts else out_type[0],
        mesh=_mesh,
        compiler_params=_sc_params,
        scratch_types=scratch,
    )
    def _segsum(z_hbm, srcl_hbm, locl_hbm, nch_hbm, out_hbm, *rest):
        if with_counts:
            (cnt_hbm, src0, src1, loc0, loc1, rows0, rows1, acc_v, nch_v,
             sem0, sem1, seml0, seml1, cnt_v) = rest
        else:
            (src0, src1, loc0, loc1, rows0, rows1, acc_v, nch_v,
             sem0, sem1, seml0, seml1) = rest
        c = lax.axis_index("c")
        s = lax.axis_index("s")
        w = c * 16 + s
        base = w * _WIN
        lbase = w * _EMAX

        zero16 = jnp.zeros((16,), jnp.float32)
        one16 = jnp.ones((16,), jnp.float32)

        def _zacc(r, carry):
            for k in range(_D // 16):
                acc_v[r, pl.ds(k * 16, 16)] = zero16
            if with_counts:
                cnt_v[pl.ds(r * _CW, _CW)] = zero16
            return carry

        lax.fori_loop(0, _AROWS, _zacc, 0)

        pltpu.sync_copy(nch_hbm.at[pl.ds(w * _CW, _CW)], nch_v)
        nch = nch_v[...][0]

        def _lstart(i, srcb, locb, seml):
            @pl.when(i < nch)
            def _():
                off = pl.multiple_of(lbase + i * _C, 16)
                pltpu.async_copy(srcl_hbm.at[pl.ds(off, _C)], srcb, seml)
                pltpu.async_copy(locl_hbm.at[pl.ds(off, _C)],
                                 locb.at[pl.ds(0, _C)], seml)

        def _lwait(i, srcb, locb, seml):
            off = pl.multiple_of(lbase + i * _C, 16)
            pltpu.make_async_copy(srcl_hbm.at[pl.ds(off, _C)], srcb,
                                  seml).wait()
            pltpu.make_async_copy(locl_hbm.at[pl.ds(off, _C)],
                                  locb.at[pl.ds(0, _C)], seml).wait()

        def _finish(srcb, rowsb, sem):
            pltpu.make_async_copy(z_hbm.at[srcb], rowsb, sem).wait()

        def _edges(rowsb, locb):
            def _grp(j4, carry):
                jb = j4 * 4
                locs = locb[pl.ds(jb, 16)]
                ls = [locs[0], locs[1], locs[2], locs[3]]
                if with_counts:
                    for l in range(4):
                        plsc.addupdate(cnt_v.at[pl.ds(ls[l] * _CW, _CW)],
                                       one16)
                for l in range(4):
                    row = jb + l
                    vals = [rowsb[row, pl.ds(k * 16, 16)]
                            for k in range(_D // 16)]
                    for k in range(_D // 16):
                        plsc.addupdate(acc_v.at[ls[l], pl.ds(k * 16, 16)],
                                       vals[k])
                return carry

            lax.fori_loop(0, _C // 4, _grp, 0)

        @pl.when(0 < nch)
        def _():
            off = pl.multiple_of(lbase, 16)
            pltpu.sync_copy(srcl_hbm.at[pl.ds(off, _C)], src0)
            pltpu.sync_copy(locl_hbm.at[pl.ds(off, _C)], loc0.at[pl.ds(0, _C)])
            pltpu.async_copy(z_hbm.at[src0], rows0, sem0)

        _lstart(1, src1, loc1, seml1)

        def _outer(g, carry):
            i0 = g * 2
            _finish(src0, rows0, sem0)

            @pl.when(i0 + 1 < nch)
            def _():
                _lwait(i0 + 1, src1, loc1, seml1)
                pltpu.async_copy(z_hbm.at[src1], rows1, sem1)

            _edges(rows0, loc0)
            _lstart(i0 + 2, src0, loc0, seml0)

            @pl.when(i0 + 1 < nch)
            def _():
                _finish(src1, rows1, sem1)

                @pl.when(i0 + 2 < nch)
                def _():
                    _lwait(i0 + 2, src0, loc0, seml0)
                    pltpu.async_copy(z_hbm.at[src0], rows0, sem0)

                _edges(rows1, loc1)
                _lstart(i0 + 3, src1, loc1, seml1)

            return carry

        lax.fori_loop(0, (nch + 1) // 2, _outer, 0)

        pltpu.sync_copy(acc_v.at[pl.ds(0, _WIN)], out_hbm.at[pl.ds(base, _WIN)])
        if with_counts:
            pltpu.sync_copy(cnt_v.at[pl.ds(0, _WIN * _CW)],
                            cnt_hbm.at[pl.ds(base * _CW, _WIN * _CW)])

    return _segsum


_segsum_c = _make_segsum(True)
_segsum_n = _make_segsum(False)


_BR = 1024


def _fused_body(s_ref, c_ref, x_ref, wl_ref, wr_ref, b_ref, o_ref, *, elu):
    cnt = jnp.maximum(c_ref[...][:, 0:1], 1.0)
    mean = s_ref[...] / cnt
    acc = lax.dot_general(mean, wl_ref[...], (((1,), (1,)), ((), ())),
                          precision=lax.Precision.HIGHEST,
                          preferred_element_type=jnp.float32)
    acc = acc + lax.dot_general(x_ref[...], wr_ref[...], (((1,), (1,)), ((), ())),
                                precision=lax.Precision.HIGHEST,
                                preferred_element_type=jnp.float32)
    acc = acc + b_ref[...]
    if elu:
        acc = jnp.where(acc > 0.0, acc, jnp.exp(jnp.minimum(acc, 0.0)) - 1.0)
    o_ref[...] = acc


def _fused(ssum, cnt, x, w_l, w_r, b, elu):
    return pl.pallas_call(
        functools.partial(_fused_body, elu=elu),
        grid=(_NPAD // _BR,),
        in_specs=[
            pl.BlockSpec((_BR, _D), lambda i: (i, 0)),
            pl.BlockSpec((_BR, _CW), lambda i: (i, 0)),
            pl.BlockSpec((_BR, _D), lambda i: (i, 0)),
            pl.BlockSpec((_D, _D), lambda i: (0, 0)),
            pl.BlockSpec((_D, _D), lambda i: (0, 0)),
            pl.BlockSpec((1, _D), lambda i: (0, 0)),
        ],
        out_specs=pl.BlockSpec((_BR, _D), lambda i: (i, 0)),
        out_shape=jax.ShapeDtypeStruct((_NPAD, _D), jnp.float32),
    )(ssum, cnt, x, w_l, w_r, b)


def kernel(x, edge_index, X_param, W1_l, b1_l, W1_r, W2_l, b2_l, W2_r):
    del x
    src = edge_index[0].astype(jnp.int32)
    dst = edge_index[1].astype(jnp.int32)
    xp = jnp.pad(X_param, ((0, _NPAD - _NODES), (0, 0)))
    srcl, locl, nch = _filter(src, dst)
    s1, cnt = _segsum_c(xp, srcl, locl, nch)
    cnt = cnt.reshape(_NPAD, _CW)
    h = _fused(s1, cnt, xp, W1_l, W1_r, b1_l.reshape(1, _D), True)
    s2 = _segsum_n(h, srcl, locl, nch)
    out = _fused(s2, cnt, h, W2_l, W2_r, b2_l.reshape(1, _D), False)
    return out[:_NODES]


# speedup vs baseline: 4.7478x; 1.1862x over previous
import functools

import jax
import jax.numpy as jnp
from jax import lax
from jax.experimental import pallas as pl
from jax.experimental.pallas import tpu as pltpu
from jax.experimental.pallas import tpu_sc as plsc

_NODES = 10000
_EDGES = 160000
_D = 256

_NW = 32
_WIN = 320
_NPAD = _NW * _WIN
_TRASH = _WIN
_AROWS = _WIN + 8
_CW = 16

_CF = 4000
_NFC = _EDGES // _CF
_C = 64
_EMAX = 160768

_mesh = plsc.VectorSubcoreMesh(core_axis_name="c", subcore_axis_name="s")
_sc_params = pltpu.CompilerParams(needs_layout_passes=False)


@functools.partial(
    pl.kernel,
    out_type=(
        jax.ShapeDtypeStruct((_NW * _EMAX,), jnp.int32),
        jax.ShapeDtypeStruct((_NW * _EMAX,), jnp.int32),
        jax.ShapeDtypeStruct((_NW * _CW,), jnp.int32),
    ),
    mesh=_mesh,
    compiler_params=_sc_params,
    scratch_types=[
        pltpu.VMEM((_CF,), jnp.int32),
        pltpu.VMEM((_CF,), jnp.int32),
        pltpu.VMEM((_CF,), jnp.int32),
        pltpu.VMEM((_CF,), jnp.int32),
        pltpu.VMEM((_CF + 16,), jnp.int32),
        pltpu.VMEM((_CF + 16,), jnp.int32),
        pltpu.VMEM((_CF + 16,), jnp.int32),
        pltpu.VMEM((_CF + 16,), jnp.int32),
        pltpu.VMEM((_CW,), jnp.int32),
        pltpu.SemaphoreType.DMA,
        pltpu.SemaphoreType.DMA,
        pltpu.SemaphoreType.DMA,
        pltpu.SemaphoreType.DMA,
    ],
)
def _filter(src_hbm, dst_hbm, srcl_hbm, locl_hbm, nch_hbm,
            fsrc0, fsrc1, fdst0, fdst1, csrc0, csrc1, cloc0, cloc1, nch_v,
            semL0, semL1, semS0, semS1):
    c = lax.axis_index("c")
    s = lax.axis_index("s")
    w = c * 16 + s
    base = w * _WIN
    lbase = w * _EMAX

    padsrc16 = jnp.full((16,), 1, jnp.int32) * base
    trash16 = jnp.full((16,), _TRASH, jnp.int32)
    win_u = jnp.uint32(_WIN)

    def _lstart(i, fsrcb, fdstb, semL):
        @pl.when(i < _NFC)
        def _():
            pltpu.async_copy(src_hbm.at[pl.ds(i * _CF, _CF)], fsrcb, semL)
            pltpu.async_copy(dst_hbm.at[pl.ds(i * _CF, _CF)], fdstb, semL)

    def _lwait(i, fsrcb, fdstb, semL):
        pltpu.make_async_copy(src_hbm.at[pl.ds(i * _CF, _CF)], fsrcb,
                              semL).wait()
        pltpu.make_async_copy(dst_hbm.at[pl.ds(i * _CF, _CF)], fdstb,
                              semL).wait()

    def _swait(off, csrcb, clocb, semS):
        off = pl.multiple_of(off, 16)
        pltpu.make_async_copy(csrcb, srcl_hbm.at[pl.ds(off, _CF + 16)],
                              semS).wait()
        pltpu.make_async_copy(clocb, locl_hbm.at[pl.ds(off, _CF + 16)],
                              semS).wait()

    def _compact(fsrcb, fdstb, csrcb, clocb):
        def _vec(j, n):
            o = j * 32
            d0 = fdstb[pl.ds(o, 16)]
            s0 = fsrcb[pl.ds(o, 16)]
            d1 = fdstb[pl.ds(o + 16, 16)]
            s1 = fsrcb[pl.ds(o + 16, 16)]
            loc0 = d0 - base
            ok0 = loc0.astype(jnp.uint32) < win_u
            plsc.store_compressed(csrcb.at[pl.ds(n, 16)], s0, mask=ok0)
            plsc.store_compressed(clocb.at[pl.ds(n, 16)], loc0, mask=ok0)
            n = n + plsc.all_reduce_population_count(ok0)[0]
            loc1 = d1 - base
            ok1 = loc1.astype(jnp.uint32) < win_u
            plsc.store_compressed(csrcb.at[pl.ds(n, 16)], s1, mask=ok1)
            plsc.store_compressed(clocb.at[pl.ds(n, 16)], loc1, mask=ok1)
            return n + plsc.all_reduce_population_count(ok1)[0]

        n = lax.fori_loop(0, _CF // 32, _vec, jnp.int32(0))
        csrcb[pl.ds(n, 16)] = padsrc16
        clocb[pl.ds(n, 16)] = trash16
        return ((n + 15) // 16) * 16

    _lstart(0, fsrc0, fdst0, semL0)
    _lstart(1, fsrc1, fdst1, semL1)

    def _scan2(g, carry):
        out_off, offB = carry
        i0 = g * 2

        _lwait(i0, fsrc0, fdst0, semL0)
        n_r0 = _compact(fsrc0, fdst0, csrc0, cloc0)

        @pl.when(g > 0)
        def _():
            _swait(offB, csrc1, cloc1, semS1)

        dst0 = pl.multiple_of(lbase + out_off, 16)
        pltpu.async_copy(csrc0, srcl_hbm.at[pl.ds(dst0, _CF + 16)], semS0)
        pltpu.async_copy(cloc0, locl_hbm.at[pl.ds(dst0, _CF + 16)], semS0)
        _lstart(i0 + 2, fsrc0, fdst0, semL0)
        out_off = out_off + n_r0

        _lwait(i0 + 1, fsrc1, fdst1, semL1)
        n_r1 = _compact(fsrc1, fdst1, csrc1, cloc1)

        _swait(dst0, csrc0, cloc0, semS0)

        dst1 = pl.multiple_of(lbase + out_off, 16)
        pltpu.async_copy(csrc1, srcl_hbm.at[pl.ds(dst1, _CF + 16)], semS1)
        pltpu.async_copy(cloc1, locl_hbm.at[pl.ds(dst1, _CF + 16)], semS1)
        _lstart(i0 + 3, fsrc1, fdst1, semL1)
        return (out_off + n_r1, dst1)

    n_tot, offB = lax.fori_loop(0, _NFC // 2, _scan2, (jnp.int32(0),
                                                       jnp.int32(lbase)))

    _swait(offB, csrc1, cloc1, semS1)

    for k in range(5):
        csrc0[pl.ds(k * 16, 16)] = padsrc16
        cloc0[pl.ds(k * 16, 16)] = trash16
    tail_off = pl.multiple_of(lbase + n_tot, 16)
    pltpu.sync_copy(csrc0.at[pl.ds(0, 80)], srcl_hbm.at[pl.ds(tail_off, 80)])
    pltpu.sync_copy(cloc0.at[pl.ds(0, 80)], locl_hbm.at[pl.ds(tail_off, 80)])

    nch = (n_tot + _C - 1) // _C
    nch_v[...] = jnp.full((_CW,), 1, jnp.int32) * nch
    pltpu.sync_copy(nch_v, nch_hbm.at[pl.ds(w * _CW, _CW)])


def _make_segsum(with_counts):
    out_type = [jax.ShapeDtypeStruct((_NPAD, _D), jnp.float32)]
    scratch = [
        pltpu.VMEM((_C,), jnp.int32),
        pltpu.VMEM((_C,), jnp.int32),
        pltpu.VMEM((_C + 16,), jnp.int32),
        pltpu.VMEM((_C + 16,), jnp.int32),
        pltpu.VMEM((_C, _D), jnp.float32),
        pltpu.VMEM((_C, _D), jnp.float32),
        pltpu.VMEM((_AROWS, _D), jnp.float32),
        pltpu.VMEM((_CW,), jnp.int32),
        pltpu.SemaphoreType.DMA,
        pltpu.SemaphoreType.DMA,
        pltpu.SemaphoreType.DMA,
        pltpu.SemaphoreType.DMA,
    ]
    if with_counts:
        out_type.append(jax.ShapeDtypeStruct((_NPAD * _CW,), jnp.float32))
        scratch.append(pltpu.VMEM((_AROWS * _CW,), jnp.float32))

    @functools.partial(
        pl.kernel,
        out_type=tuple(out_type) if with_counts else out_type[0],
        mesh=_mesh,
        compiler_params=_sc_params,
        scratch_types=scratch,
    )
    def _segsum(z_hbm, srcl_hbm, locl_hbm, nch_hbm, out_hbm, *rest):
        if with_counts:
            (cnt_hbm, src0, src1, loc0, loc1, rows0, rows1, acc_v, nch_v,
             sem0, sem1, seml0, seml1, cnt_v) = rest
        else:
            (src0, src1, loc0, loc1, rows0, rows1, acc_v, nch_v,
             sem0, sem1, seml0, seml1) = rest
        c = lax.axis_index("c")
        s = lax.axis_index("s")
        w = c * 16 + s
        base = w * _WIN
        lbase = w * _EMAX

        zero16 = jnp.zeros((16,), jnp.float32)
        one16 = jnp.ones((16,), jnp.float32)

        def _zacc(r, carry):
            for k in range(_D // 16):
                acc_v[r, pl.ds(k * 16, 16)] = zero16
            if with_counts:
                cnt_v[pl.ds(r * _CW, _CW)] = zero16
            return carry

        lax.fori_loop(0, _AROWS, _zacc, 0)

        pltpu.sync_copy(nch_hbm.at[pl.ds(w * _CW, _CW)], nch_v)
        nch = nch_v[...][0]

        def _lstart(i, srcb, locb, seml):
            @pl.when(i < nch)
            def _():
                off = pl.multiple_of(lbase + i * _C, 16)
                pltpu.async_copy(srcl_hbm.at[pl.ds(off, _C)], srcb, seml)
                pltpu.async_copy(locl_hbm.at[pl.ds(off, _C)],
                                 locb.at[pl.ds(0, _C)], seml)

        def _lwait(i, srcb, locb, seml):
            off = pl.multiple_of(lbase + i * _C, 16)
            pltpu.make_async_copy(srcl_hbm.at[pl.ds(off, _C)], srcb,
                                  seml).wait()
            pltpu.make_async_copy(locl_hbm.at[pl.ds(off, _C)],
                                  locb.at[pl.ds(0, _C)], seml).wait()

        def _finish(srcb, rowsb, sem):
            pltpu.make_async_copy(z_hbm.at[srcb], rowsb, sem).wait()

        def _edges(rowsb, locb):
            def _grp(j4, carry):
                jb = j4 * 4
                locs = locb[pl.ds(jb, 16)]
                ls = [locs[0], locs[1], locs[2], locs[3]]
                if with_counts:
                    for l in range(4):
                        plsc.addupdate(cnt_v.at[pl.ds(ls[l] * _CW, _CW)],
                                       one16)
                for l in range(4):
                    row = jb + l
                    vals = [rowsb[row, pl.ds(k * 16, 16)]
                            for k in range(_D // 16)]
                    for k in range(_D // 16):
                        plsc.addupdate(acc_v.at[ls[l], pl.ds(k * 16, 16)],
                                       vals[k])
                return carry

            lax.fori_loop(0, _C // 4, _grp, 0)

        @pl.when(0 < nch)
        def _():
            off = pl.multiple_of(lbase, 16)
            pltpu.sync_copy(srcl_hbm.at[pl.ds(off, _C)], src0)
            pltpu.sync_copy(locl_hbm.at[pl.ds(off, _C)], loc0.at[pl.ds(0, _C)])
            pltpu.async_copy(z_hbm.at[src0], rows0, sem0)

        _lstart(1, src1, loc1, seml1)

        def _outer(g, carry):
            i0 = g * 2
            _finish(src0, rows0, sem0)

            @pl.when(i0 + 1 < nch)
            def _():
                _lwait(i0 + 1, src1, loc1, seml1)
                pltpu.async_copy(z_hbm.at[src1], rows1, sem1)

            _edges(rows0, loc0)
            _lstart(i0 + 2, src0, loc0, seml0)

            @pl.when(i0 + 1 < nch)
            def _():
                _finish(src1, rows1, sem1)

                @pl.when(i0 + 2 < nch)
                def _():
                    _lwait(i0 + 2, src0, loc0, seml0)
                    pltpu.async_copy(z_hbm.at[src0], rows0, sem0)

                _edges(rows1, loc1)
                _lstart(i0 + 3, src1, loc1, seml1)

            return carry

        lax.fori_loop(0, (nch + 1) // 2, _outer, 0)

        pltpu.sync_copy(acc_v.at[pl.ds(0, _WIN)], out_hbm.at[pl.ds(base, _WIN)])
        if with_counts:
            pltpu.sync_copy(cnt_v.at[pl.ds(0, _WIN * _CW)],
                            cnt_hbm.at[pl.ds(base * _CW, _WIN * _CW)])

    return _segsum


_segsum_c = _make_segsum(True)
_segsum_n = _make_segsum(False)


_BR = 1024


def _fused_body(s_ref, c_ref, x_ref, wl_ref, wr_ref, b_ref, o_ref, *, elu):
    cnt = jnp.maximum(c_ref[...][:, 0:1], 1.0)
    mean = s_ref[...] / cnt
    acc = lax.dot_general(mean, wl_ref[...], (((1,), (1,)), ((), ())),
                          precision=lax.Precision.HIGHEST,
                          preferred_element_type=jnp.float32)
    acc = acc + lax.dot_general(x_ref[...], wr_ref[...], (((1,), (1,)), ((), ())),
                                precision=lax.Precision.HIGHEST,
                                preferred_element_type=jnp.float32)
    acc = acc + b_ref[...]
    if elu:
        acc = jnp.where(acc > 0.0, acc, jnp.exp(jnp.minimum(acc, 0.0)) - 1.0)
    o_ref[...] = acc


def _fused(ssum, cnt, x, w_l, w_r, b, elu):
    return pl.pallas_call(
        functools.partial(_fused_body, elu=elu),
        grid=(_NPAD // _BR,),
        in_specs=[
            pl.BlockSpec((_BR, _D), lambda i: (i, 0)),
            pl.BlockSpec((_BR, _CW), lambda i: (i, 0)),
            pl.BlockSpec((_BR, _D), lambda i: (i, 0)),
            pl.BlockSpec((_D, _D), lambda i: (0, 0)),
            pl.BlockSpec((_D, _D), lambda i: (0, 0)),
            pl.BlockSpec((1, _D), lambda i: (0, 0)),
        ],
        out_specs=pl.BlockSpec((_BR, _D), lambda i: (i, 0)),
        out_shape=jax.ShapeDtypeStruct((_NPAD, _D), jnp.float32),
    )(ssum, cnt, x, w_l, w_r, b)


def kernel(x, edge_index, X_param, W1_l, b1_l, W1_r, W2_l, b2_l, W2_r):
    del x
    src = edge_index[0].astype(jnp.int32)
    dst = edge_index[1].astype(jnp.int32)
    xp = jnp.pad(X_param, ((0, _NPAD - _NODES), (0, 0)))
    srcl, locl, nch = _filter(src, dst)
    s1, cnt = _segsum_c(xp, srcl, locl, nch)
    cnt = cnt.reshape(_NPAD, _CW)
    h = _fused(s1, cnt, xp, W1_l, W1_r, b1_l.reshape(1, _D), True)
    s2 = _segsum_n(h, srcl, locl, nch)
    out = _fused(s2, cnt, h, W2_l, W2_r, b2_l.reshape(1, _D), False)
    return out[:_NODES]


# speedup vs baseline: 4.8111x; 1.0133x over previous
import functools

import jax
import jax.numpy as jnp
from jax import lax
from jax.experimental import pallas as pl
from jax.experimental.pallas import tpu as pltpu
from jax.experimental.pallas import tpu_sc as plsc

_NODES = 10000
_EDGES = 160000
_D = 256

_NW = 32
_WIN = 320
_NPAD = _NW * _WIN
_TRASH = _WIN
_AROWS = _WIN + 8
_CW = 16

_CF = 4000
_NFC = _EDGES // _CF
_C = 64
_EMAX = 160768

_mesh = plsc.VectorSubcoreMesh(core_axis_name="c", subcore_axis_name="s")
_sc_params = pltpu.CompilerParams(needs_layout_passes=False)


@functools.partial(
    pl.kernel,
    out_type=(
        jax.ShapeDtypeStruct((_NW * _EMAX,), jnp.int32),
        jax.ShapeDtypeStruct((_NW * _EMAX,), jnp.int32),
        jax.ShapeDtypeStruct((_NW * _CW,), jnp.int32),
    ),
    mesh=_mesh,
    compiler_params=_sc_params,
    scratch_types=[
        pltpu.VMEM((_CF,), jnp.int32),
        pltpu.VMEM((_CF,), jnp.int32),
        pltpu.VMEM((_CF,), jnp.int32),
        pltpu.VMEM((_CF,), jnp.int32),
        pltpu.VMEM((_CF + 16,), jnp.int32),
        pltpu.VMEM((_CF + 16,), jnp.int32),
        pltpu.VMEM((_CF + 16,), jnp.int32),
        pltpu.VMEM((_CF + 16,), jnp.int32),
        pltpu.VMEM((_CW,), jnp.int32),
        pltpu.SemaphoreType.DMA,
        pltpu.SemaphoreType.DMA,
        pltpu.SemaphoreType.DMA,
        pltpu.SemaphoreType.DMA,
    ],
)
def _filter(src_hbm, dst_hbm, srcl_hbm, locl_hbm, nch_hbm,
            fsrc0, fsrc1, fdst0, fdst1, csrc0, csrc1, cloc0, cloc1, nch_v,
            semL0, semL1, semS0, semS1):
    c = lax.axis_index("c")
    s = lax.axis_index("s")
    w = c * 16 + s
    base = w * _WIN
    lbase = w * _EMAX

    padsrc16 = jnp.full((16,), 1, jnp.int32) * base
    trash16 = jnp.full((16,), _TRASH, jnp.int32)
    win_u = jnp.uint32(_WIN)

    def _lstart(i, fsrcb, fdstb, semL):
        @pl.when(i < _NFC)
        def _():
            pltpu.async_copy(src_hbm.at[pl.ds(i * _CF, _CF)], fsrcb, semL)
            pltpu.async_copy(dst_hbm.at[pl.ds(i * _CF, _CF)], fdstb, semL)

    def _lwait(i, fsrcb, fdstb, semL):
        pltpu.make_async_copy(src_hbm.at[pl.ds(i * _CF, _CF)], fsrcb,
                              semL).wait()
        pltpu.make_async_copy(dst_hbm.at[pl.ds(i * _CF, _CF)], fdstb,
                              semL).wait()

    def _swait(off, csrcb, clocb, semS):
        off = pl.multiple_of(off, 16)
        pltpu.make_async_copy(csrcb, srcl_hbm.at[pl.ds(off, _CF + 16)],
                              semS).wait()
        pltpu.make_async_copy(clocb, locl_hbm.at[pl.ds(off, _CF + 16)],
                              semS).wait()

    def _compact(fsrcb, fdstb, csrcb, clocb):
        def _vec(j, n):
            o = j * 32
            d0 = fdstb[pl.ds(o, 16)]
            s0 = fsrcb[pl.ds(o, 16)]
            d1 = fdstb[pl.ds(o + 16, 16)]
            s1 = fsrcb[pl.ds(o + 16, 16)]
            loc0 = d0 - base
            ok0 = loc0.astype(jnp.uint32) < win_u
            plsc.store_compressed(csrcb.at[pl.ds(n, 16)], s0, mask=ok0)
            plsc.store_compressed(clocb.at[pl.ds(n, 16)], loc0, mask=ok0)
            n = n + plsc.all_reduce_population_count(ok0)[0]
            loc1 = d1 - base
            ok1 = loc1.astype(jnp.uint32) < win_u
            plsc.store_compressed(csrcb.at[pl.ds(n, 16)], s1, mask=ok1)
            plsc.store_compressed(clocb.at[pl.ds(n, 16)], loc1, mask=ok1)
            return n + plsc.all_reduce_population_count(ok1)[0]

        n = lax.fori_loop(0, _CF // 32, _vec, jnp.int32(0))
        csrcb[pl.ds(n, 16)] = padsrc16
        clocb[pl.ds(n, 16)] = trash16
        return ((n + 15) // 16) * 16

    _lstart(0, fsrc0, fdst0, semL0)
    _lstart(1, fsrc1, fdst1, semL1)

    def _scan2(g, carry):
        out_off, offB = carry
        i0 = g * 2

        _lwait(i0, fsrc0, fdst0, semL0)
        n_r0 = _compact(fsrc0, fdst0, csrc0, cloc0)

        @pl.when(g > 0)
        def _():
            _swait(offB, csrc1, cloc1, semS1)

        dst0 = pl.multiple_of(lbase + out_off, 16)
        pltpu.async_copy(csrc0, srcl_hbm.at[pl.ds(dst0, _CF + 16)], semS0)
        pltpu.async_copy(cloc0, locl_hbm.at[pl.ds(dst0, _CF + 16)], semS0)
        _lstart(i0 + 2, fsrc0, fdst0, semL0)
        out_off = out_off + n_r0

        _lwait(i0 + 1, fsrc1, fdst1, semL1)
        n_r1 = _compact(fsrc1, fdst1, csrc1, cloc1)

        _swait(dst0, csrc0, cloc0, semS0)

        dst1 = pl.multiple_of(lbase + out_off, 16)
        pltpu.async_copy(csrc1, srcl_hbm.at[pl.ds(dst1, _CF + 16)], semS1)
        pltpu.async_copy(cloc1, locl_hbm.at[pl.ds(dst1, _CF + 16)], semS1)
        _lstart(i0 + 3, fsrc1, fdst1, semL1)
        return (out_off + n_r1, dst1)

    n_tot, offB = lax.fori_loop(0, _NFC // 2, _scan2, (jnp.int32(0),
                                                       jnp.int32(lbase)))

    _swait(offB, csrc1, cloc1, semS1)

    for k in range(5):
        csrc0[pl.ds(k * 16, 16)] = padsrc16
        cloc0[pl.ds(k * 16, 16)] = trash16
    tail_off = pl.multiple_of(lbase + n_tot, 16)
    pltpu.sync_copy(csrc0.at[pl.ds(0, 80)], srcl_hbm.at[pl.ds(tail_off, 80)])
    pltpu.sync_copy(cloc0.at[pl.ds(0, 80)], locl_hbm.at[pl.ds(tail_off, 80)])

    nch = (n_tot + _C - 1) // _C
    nch_v[...] = jnp.full((_CW,), 1, jnp.int32) * nch
    pltpu.sync_copy(nch_v, nch_hbm.at[pl.ds(w * _CW, _CW)])


def _make_segsum(with_counts):
    out_type = [jax.ShapeDtypeStruct((_NPAD, _D), jnp.float32)]
    scratch = [
        pltpu.VMEM((_C,), jnp.int32),
        pltpu.VMEM((_C,), jnp.int32),
        pltpu.VMEM((_C + 16,), jnp.int32),
        pltpu.VMEM((_C + 16,), jnp.int32),
        pltpu.VMEM((_C, _D), jnp.float32),
        pltpu.VMEM((_C, _D), jnp.float32),
        pltpu.VMEM((_AROWS, _D), jnp.float32),
        pltpu.VMEM((_CW,), jnp.int32),
        pltpu.SemaphoreType.DMA,
        pltpu.SemaphoreType.DMA,
        pltpu.SemaphoreType.DMA,
        pltpu.SemaphoreType.DMA,
    ]
    if with_counts:
        out_type.append(jax.ShapeDtypeStruct((_NPAD * _CW,), jnp.float32))
        scratch.append(pltpu.VMEM((_AROWS * _CW,), jnp.float32))

    @functools.partial(
        pl.kernel,
        out_type=tuple(out_type) if with_counts else out_type[0],
        mesh=_mesh,
        compiler_params=_sc_params,
        scratch_types=scratch,
    )
    def _segsum(z_hbm, srcl_hbm, locl_hbm, nch_hbm, out_hbm, *rest):
        if with_counts:
            (cnt_hbm, src0, src1, loc0, loc1, rows0, rows1, acc_v, nch_v,
             sem0, sem1, seml0, seml1, cnt_v) = rest
        else:
            (src0, src1, loc0, loc1, rows0, rows1, acc_v, nch_v,
             sem0, sem1, seml0, seml1) = rest
        c = lax.axis_index("c")
        s = lax.axis_index("s")
        w = c * 16 + s
        base = w * _WIN
        lbase = w * _EMAX

        zero16 = jnp.zeros((16,), jnp.float32)
        one16 = jnp.ones((16,), jnp.float32)

        def _zacc(r, carry):
            for k in range(_D // 16):
                acc_v[r, pl.ds(k * 16, 16)] = zero16
            if with_counts:
                cnt_v[pl.ds(r * _CW, _CW)] = zero16
            return carry

        lax.fori_loop(0, _AROWS, _zacc, 0)

        pltpu.sync_copy(nch_hbm.at[pl.ds(w * _CW, _CW)], nch_v)
        nch = nch_v[...][0]

        def _lstart(i, srcb, locb, seml):
            @pl.when(i < nch)
            def _():
                off = pl.multiple_of(lbase + i * _C, 16)
                pltpu.async_copy(srcl_hbm.at[pl.ds(off, _C)], srcb, seml)
                pltpu.async_copy(locl_hbm.at[pl.ds(off, _C)],
                                 locb.at[pl.ds(0, _C)], seml)

        def _lwait(i, srcb, locb, seml):
            off = pl.multiple_of(lbase + i * _C, 16)
            pltpu.make_async_copy(srcl_hbm.at[pl.ds(off, _C)], srcb,
                                  seml).wait()
            pltpu.make_async_copy(locl_hbm.at[pl.ds(off, _C)],
                                  locb.at[pl.ds(0, _C)], seml).wait()

        def _finish(srcb, rowsb, sem):
            pltpu.make_async_copy(z_hbm.at[srcb], rowsb, sem).wait()

        def _edges(rowsb, locb):
            def _grp(j8, carry):
                jb = j8 * 8
                locs = locb[pl.ds(jb, 16)]
                ls = [locs[l] for l in range(8)]
                if with_counts:
                    for l in range(8):
                        plsc.addupdate(cnt_v.at[pl.ds(ls[l] * _CW, _CW)],
                                       one16)
                for l in range(8):
                    row = jb + l
                    vals = [rowsb[row, pl.ds(k * 16, 16)]
                            for k in range(_D // 16)]
                    for k in range(_D // 16):
                        plsc.addupdate(acc_v.at[ls[l], pl.ds(k * 16, 16)],
                                       vals[k])
                return carry

            lax.fori_loop(0, _C // 8, _grp, 0)

        @pl.when(0 < nch)
        def _():
            off = pl.multiple_of(lbase, 16)
            pltpu.sync_copy(srcl_hbm.at[pl.ds(off, _C)], src0)
            pltpu.sync_copy(locl_hbm.at[pl.ds(off, _C)], loc0.at[pl.ds(0, _C)])
            pltpu.async_copy(z_hbm.at[src0], rows0, sem0)

        _lstart(1, src1, loc1, seml1)

        def _outer(g, carry):
            i0 = g * 2
            _finish(src0, rows0, sem0)

            @pl.when(i0 + 1 < nch)
            def _():
                _lwait(i0 + 1, src1, loc1, seml1)
                pltpu.async_copy(z_hbm.at[src1], rows1, sem1)

            _edges(rows0, loc0)
            _lstart(i0 + 2, src0, loc0, seml0)

            @pl.when(i0 + 1 < nch)
            def _():
                _finish(src1, rows1, sem1)

                @pl.when(i0 + 2 < nch)
                def _():
                    _lwait(i0 + 2, src0, loc0, seml0)
                    pltpu.async_copy(z_hbm.at[src0], rows0, sem0)

                _edges(rows1, loc1)
                _lstart(i0 + 3, src1, loc1, seml1)

            return carry

        lax.fori_loop(0, (nch + 1) // 2, _outer, 0)

        pltpu.sync_copy(acc_v.at[pl.ds(0, _WIN)], out_hbm.at[pl.ds(base, _WIN)])
        if with_counts:
            pltpu.sync_copy(cnt_v.at[pl.ds(0, _WIN * _CW)],
                            cnt_hbm.at[pl.ds(base * _CW, _WIN * _CW)])

    return _segsum


_segsum_c = _make_segsum(True)
_segsum_n = _make_segsum(False)


_BR = 1024


def _fused_body(s_ref, c_ref, x_ref, wl_ref, wr_ref, b_ref, o_ref, *, elu):
    cnt = jnp.maximum(c_ref[...][:, 0:1], 1.0)
    mean = s_ref[...] / cnt
    acc = lax.dot_general(mean, wl_ref[...], (((1,), (1,)), ((), ())),
                          precision=lax.Precision.HIGHEST,
                          preferred_element_type=jnp.float32)
    acc = acc + lax.dot_general(x_ref[...], wr_ref[...], (((1,), (1,)), ((), ())),
                                precision=lax.Precision.HIGHEST,
                                preferred_element_type=jnp.float32)
    acc = acc + b_ref[...]
    if elu:
        acc = jnp.where(acc > 0.0, acc, jnp.exp(jnp.minimum(acc, 0.0)) - 1.0)
    o_ref[...] = acc


def _fused(ssum, cnt, x, w_l, w_r, b, elu):
    return pl.pallas_call(
        functools.partial(_fused_body, elu=elu),
        grid=(_NPAD // _BR,),
        in_specs=[
            pl.BlockSpec((_BR, _D), lambda i: (i, 0)),
            pl.BlockSpec((_BR, _CW), lambda i: (i, 0)),
            pl.BlockSpec((_BR, _D), lambda i: (i, 0)),
            pl.BlockSpec((_D, _D), lambda i: (0, 0)),
            pl.BlockSpec((_D, _D), lambda i: (0, 0)),
            pl.BlockSpec((1, _D), lambda i: (0, 0)),
        ],
        out_specs=pl.BlockSpec((_BR, _D), lambda i: (i, 0)),
        out_shape=jax.ShapeDtypeStruct((_NPAD, _D), jnp.float32),
    )(ssum, cnt, x, w_l, w_r, b)


def kernel(x, edge_index, X_param, W1_l, b1_l, W1_r, W2_l, b2_l, W2_r):
    del x
    src = edge_index[0].astype(jnp.int32)
    dst = edge_index[1].astype(jnp.int32)
    xp = jnp.pad(X_param, ((0, _NPAD - _NODES), (0, 0)))
    srcl, locl, nch = _filter(src, dst)
    s1, cnt = _segsum_c(xp, srcl, locl, nch)
    cnt = cnt.reshape(_NPAD, _CW)
    h = _fused(s1, cnt, xp, W1_l, W1_r, b1_l.reshape(1, _D), True)
    s2 = _segsum_n(h, srcl, locl, nch)
    out = _fused(s2, cnt, h, W2_l, W2_r, b2_l.reshape(1, _D), False)
    return out[:_NODES]


# speedup vs baseline: 4.9536x; 1.0296x over previous
import functools

import jax
import jax.numpy as jnp
from jax import lax
from jax.experimental import pallas as pl
from jax.experimental.pallas import tpu as pltpu
from jax.experimental.pallas import tpu_sc as plsc

_NODES = 10000
_EDGES = 160000
_D = 256

_NW = 32
_WIN = 320
_NPAD = _NW * _WIN
_TRASH = _WIN
_AROWS = _WIN + 8
_CW = 16

_CF = 8000
_NFC = _EDGES // _CF
_C = 64
_EMAX = 160768

_mesh = plsc.VectorSubcoreMesh(core_axis_name="c", subcore_axis_name="s")
_sc_params = pltpu.CompilerParams(needs_layout_passes=False)


@functools.partial(
    pl.kernel,
    out_type=(
        jax.ShapeDtypeStruct((_NW * _EMAX,), jnp.int32),
        jax.ShapeDtypeStruct((_NW * _EMAX,), jnp.int32),
        jax.ShapeDtypeStruct((_NW * _CW,), jnp.int32),
    ),
    mesh=_mesh,
    compiler_params=_sc_params,
    scratch_types=[
        pltpu.VMEM((_CF,), jnp.int32),
        pltpu.VMEM((_CF,), jnp.int32),
        pltpu.VMEM((_CF,), jnp.int32),
        pltpu.VMEM((_CF,), jnp.int32),
        pltpu.VMEM((_CF + 16,), jnp.int32),
        pltpu.VMEM((_CF + 16,), jnp.int32),
        pltpu.VMEM((_CF + 16,), jnp.int32),
        pltpu.VMEM((_CF + 16,), jnp.int32),
        pltpu.VMEM((_CW,), jnp.int32),
        pltpu.SemaphoreType.DMA,
        pltpu.SemaphoreType.DMA,
        pltpu.SemaphoreType.DMA,
        pltpu.SemaphoreType.DMA,
    ],
)
def _filter(src_hbm, dst_hbm, srcl_hbm, locl_hbm, nch_hbm,
            fsrc0, fsrc1, fdst0, fdst1, csrc0, csrc1, cloc0, cloc1, nch_v,
            semL0, semL1, semS0, semS1):
    c = lax.axis_index("c")
    s = lax.axis_index("s")
    w = c * 16 + s
    base = w * _WIN
    lbase = w * _EMAX

    padsrc16 = jnp.full((16,), 1, jnp.int32) * base
    trash16 = jnp.full((16,), _TRASH, jnp.int32)
    win_u = jnp.uint32(_WIN)

    def _lstart(i, fsrcb, fdstb, semL):
        @pl.when(i < _NFC)
        def _():
            pltpu.async_copy(src_hbm.at[pl.ds(i * _CF, _CF)], fsrcb, semL)
            pltpu.async_copy(dst_hbm.at[pl.ds(i * _CF, _CF)], fdstb, semL)

    def _lwait(i, fsrcb, fdstb, semL):
        pltpu.make_async_copy(src_hbm.at[pl.ds(i * _CF, _CF)], fsrcb,
                              semL).wait()
        pltpu.make_async_copy(dst_hbm.at[pl.ds(i * _CF, _CF)], fdstb,
                              semL).wait()

    def _swait(off, csrcb, clocb, semS):
        off = pl.multiple_of(off, 16)
        pltpu.make_async_copy(csrcb, srcl_hbm.at[pl.ds(off, _CF + 16)],
                              semS).wait()
        pltpu.make_async_copy(clocb, locl_hbm.at[pl.ds(off, _CF + 16)],
                              semS).wait()

    def _compact(fsrcb, fdstb, csrcb, clocb):
        def _vec(j, n):
            o = j * 32
            d0 = fdstb[pl.ds(o, 16)]
            s0 = fsrcb[pl.ds(o, 16)]
            d1 = fdstb[pl.ds(o + 16, 16)]
            s1 = fsrcb[pl.ds(o + 16, 16)]
            loc0 = d0 - base
            ok0 = loc0.astype(jnp.uint32) < win_u
            plsc.store_compressed(csrcb.at[pl.ds(n, 16)], s0, mask=ok0)
            plsc.store_compressed(clocb.at[pl.ds(n, 16)], loc0, mask=ok0)
            n = n + plsc.all_reduce_population_count(ok0)[0]
            loc1 = d1 - base
            ok1 = loc1.astype(jnp.uint32) < win_u
            plsc.store_compressed(csrcb.at[pl.ds(n, 16)], s1, mask=ok1)
            plsc.store_compressed(clocb.at[pl.ds(n, 16)], loc1, mask=ok1)
            return n + plsc.all_reduce_population_count(ok1)[0]

        n = lax.fori_loop(0, _CF // 32, _vec, jnp.int32(0))
        csrcb[pl.ds(n, 16)] = padsrc16
        clocb[pl.ds(n, 16)] = trash16
        return ((n + 15) // 16) * 16

    _lstart(0, fsrc0, fdst0, semL0)
    _lstart(1, fsrc1, fdst1, semL1)

    def _scan2(g, carry):
        out_off, offB = carry
        i0 = g * 2

        _lwait(i0, fsrc0, fdst0, semL0)
        n_r0 = _compact(fsrc0, fdst0, csrc0, cloc0)

        @pl.when(g > 0)
        def _():
            _swait(offB, csrc1, cloc1, semS1)

        dst0 = pl.multiple_of(lbase + out_off, 16)
        pltpu.async_copy(csrc0, srcl_hbm.at[pl.ds(dst0, _CF + 16)], semS0)
        pltpu.async_copy(cloc0, locl_hbm.at[pl.ds(dst0, _CF + 16)], semS0)
        _lstart(i0 + 2, fsrc0, fdst0, semL0)
        out_off = out_off + n_r0

        _lwait(i0 + 1, fsrc1, fdst1, semL1)
        n_r1 = _compact(fsrc1, fdst1, csrc1, cloc1)

        _swait(dst0, csrc0, cloc0, semS0)

        dst1 = pl.multiple_of(lbase + out_off, 16)
        pltpu.async_copy(csrc1, srcl_hbm.at[pl.ds(dst1, _CF + 16)], semS1)
        pltpu.async_copy(cloc1, locl_hbm.at[pl.ds(dst1, _CF + 16)], semS1)
        _lstart(i0 + 3, fsrc1, fdst1, semL1)
        return (out_off + n_r1, dst1)

    n_tot, offB = lax.fori_loop(0, _NFC // 2, _scan2, (jnp.int32(0),
                                                       jnp.int32(lbase)))

    _swait(offB, csrc1, cloc1, semS1)

    for k in range(5):
        csrc0[pl.ds(k * 16, 16)] = padsrc16
        cloc0[pl.ds(k * 16, 16)] = trash16
    tail_off = pl.multiple_of(lbase + n_tot, 16)
    pltpu.sync_copy(csrc0.at[pl.ds(0, 80)], srcl_hbm.at[pl.ds(tail_off, 80)])
    pltpu.sync_copy(cloc0.at[pl.ds(0, 80)], locl_hbm.at[pl.ds(tail_off, 80)])

    nch = (n_tot + _C - 1) // _C
    nch_v[...] = jnp.full((_CW,), 1, jnp.int32) * nch
    pltpu.sync_copy(nch_v, nch_hbm.at[pl.ds(w * _CW, _CW)])


def _make_segsum(with_counts):
    out_type = [jax.ShapeDtypeStruct((_NPAD, _D), jnp.float32)]
    scratch = [
        pltpu.VMEM((_C,), jnp.int32),
        pltpu.VMEM((_C,), jnp.int32),
        pltpu.VMEM((_C + 16,), jnp.int32),
        pltpu.VMEM((_C + 16,), jnp.int32),
        pltpu.VMEM((_C, _D), jnp.float32),
        pltpu.VMEM((_C, _D), jnp.float32),
        pltpu.VMEM((_AROWS, _D), jnp.float32),
        pltpu.VMEM((_CW,), jnp.int32),
        pltpu.SemaphoreType.DMA,
        pltpu.SemaphoreType.DMA,
        pltpu.SemaphoreType.DMA,
        pltpu.SemaphoreType.DMA,
    ]
    if with_counts:
        out_type.append(jax.ShapeDtypeStruct((_NPAD * _CW,), jnp.float32))
        scratch.append(pltpu.VMEM((_AROWS * _CW,), jnp.float32))

    @functools.partial(
        pl.kernel,
        out_type=tuple(out_type) if with_counts else out_type[0],
        mesh=_mesh,
        compiler_params=_sc_params,
        scratch_types=scratch,
    )
    def _segsum(z_hbm, srcl_hbm, locl_hbm, nch_hbm, out_hbm, *rest):
        if with_counts:
            (cnt_hbm, src0, src1, loc0, loc1, rows0, rows1, acc_v, nch_v,
             sem0, sem1, seml0, seml1, cnt_v) = rest
        else:
            (src0, src1, loc0, loc1, rows0, rows1, acc_v, nch_v,
             sem0, sem1, seml0, seml1) = rest
        c = lax.axis_index("c")
        s = lax.axis_index("s")
        w = c * 16 + s
        base = w * _WIN
        lbase = w * _EMAX

        zero16 = jnp.zeros((16,), jnp.float32)
        one16 = jnp.ones((16,), jnp.float32)

        def _zacc(r, carry):
            for k in range(_D // 16):
                acc_v[r, pl.ds(k * 16, 16)] = zero16
            if with_counts:
                cnt_v[pl.ds(r * _CW, _CW)] = zero16
            return carry

        lax.fori_loop(0, _AROWS, _zacc, 0)

        pltpu.sync_copy(nch_hbm.at[pl.ds(w * _CW, _CW)], nch_v)
        nch = nch_v[...][0]

        def _lstart(i, srcb, locb, seml):
            @pl.when(i < nch)
            def _():
                off = pl.multiple_of(lbase + i * _C, 16)
                pltpu.async_copy(srcl_hbm.at[pl.ds(off, _C)], srcb, seml)
                pltpu.async_copy(locl_hbm.at[pl.ds(off, _C)],
                                 locb.at[pl.ds(0, _C)], seml)

        def _lwait(i, srcb, locb, seml):
            off = pl.multiple_of(lbase + i * _C, 16)
            pltpu.make_async_copy(srcl_hbm.at[pl.ds(off, _C)], srcb,
                                  seml).wait()
            pltpu.make_async_copy(locl_hbm.at[pl.ds(off, _C)],
                                  locb.at[pl.ds(0, _C)], seml).wait()

        def _finish(srcb, rowsb, sem):
            pltpu.make_async_copy(z_hbm.at[srcb], rowsb, sem).wait()

        def _edges(rowsb, locb):
            def _grp(j8, carry):
                jb = j8 * 8
                locs = locb[pl.ds(jb, 16)]
                ls = [locs[l] for l in range(8)]
                if with_counts:
                    for l in range(8):
                        plsc.addupdate(cnt_v.at[pl.ds(ls[l] * _CW, _CW)],
                                       one16)
                for l in range(8):
                    row = jb + l
                    vals = [rowsb[row, pl.ds(k * 16, 16)]
                            for k in range(_D // 16)]
                    for k in range(_D // 16):
                        plsc.addupdate(acc_v.at[ls[l], pl.ds(k * 16, 16)],
                                       vals[k])
                return carry

            lax.fori_loop(0, _C // 8, _grp, 0)

        @pl.when(0 < nch)
        def _():
            off = pl.multiple_of(lbase, 16)
            pltpu.sync_copy(srcl_hbm.at[pl.ds(off, _C)], src0)
            pltpu.sync_copy(locl_hbm.at[pl.ds(off, _C)], loc0.at[pl.ds(0, _C)])
            pltpu.async_copy(z_hbm.at[src0], rows0, sem0)

        _lstart(1, src1, loc1, seml1)

        def _outer(g, carry):
            i0 = g * 2
            _finish(src0, rows0, sem0)

            @pl.when(i0 + 1 < nch)
            def _():
                _lwait(i0 + 1, src1, loc1, seml1)
                pltpu.async_copy(z_hbm.at[src1], rows1, sem1)

            _edges(rows0, loc0)
            _lstart(i0 + 2, src0, loc0, seml0)

            @pl.when(i0 + 1 < nch)
            def _():
                _finish(src1, rows1, sem1)

                @pl.when(i0 + 2 < nch)
                def _():
                    _lwait(i0 + 2, src0, loc0, seml0)
                    pltpu.async_copy(z_hbm.at[src0], rows0, sem0)

                _edges(rows1, loc1)
                _lstart(i0 + 3, src1, loc1, seml1)

            return carry

        lax.fori_loop(0, (nch + 1) // 2, _outer, 0)

        pltpu.sync_copy(acc_v.at[pl.ds(0, _WIN)], out_hbm.at[pl.ds(base, _WIN)])
        if with_counts:
            pltpu.sync_copy(cnt_v.at[pl.ds(0, _WIN * _CW)],
                            cnt_hbm.at[pl.ds(base * _CW, _WIN * _CW)])

    return _segsum


_segsum_c = _make_segsum(True)
_segsum_n = _make_segsum(False)


_BR = 1024


def _fused_body(s_ref, c_ref, x_ref, wl_ref, wr_ref, b_ref, o_ref, *, elu):
    cnt = jnp.maximum(c_ref[...][:, 0:1], 1.0)
    mean = s_ref[...] / cnt
    acc = lax.dot_general(mean, wl_ref[...], (((1,), (1,)), ((), ())),
                          precision=lax.Precision.HIGHEST,
                          preferred_element_type=jnp.float32)
    acc = acc + lax.dot_general(x_ref[...], wr_ref[...], (((1,), (1,)), ((), ())),
                                precision=lax.Precision.HIGHEST,
                                preferred_element_type=jnp.float32)
    acc = acc + b_ref[...]
    if elu:
        acc = jnp.where(acc > 0.0, acc, jnp.exp(jnp.minimum(acc, 0.0)) - 1.0)
    o_ref[...] = acc


def _fused(ssum, cnt, x, w_l, w_r, b, elu):
    return pl.pallas_call(
        functools.partial(_fused_body, elu=elu),
        grid=(_NPAD // _BR,),
        in_specs=[
            pl.BlockSpec((_BR, _D), lambda i: (i, 0)),
            pl.BlockSpec((_BR, _CW), lambda i: (i, 0)),
            pl.BlockSpec((_BR, _D), lambda i: (i, 0)),
            pl.BlockSpec((_D, _D), lambda i: (0, 0)),
            pl.BlockSpec((_D, _D), lambda i: (0, 0)),
            pl.BlockSpec((1, _D), lambda i: (0, 0)),
        ],
        out_specs=pl.BlockSpec((_BR, _D), lambda i: (i, 0)),
        out_shape=jax.ShapeDtypeStruct((_NPAD, _D), jnp.float32),
    )(ssum, cnt, x, w_l, w_r, b)


def kernel(x, edge_index, X_param, W1_l, b1_l, W1_r, W2_l, b2_l, W2_r):
    del x
    src = edge_index[0].astype(jnp.int32)
    dst = edge_index[1].astype(jnp.int32)
    xp = jnp.pad(X_param, ((0, _NPAD - _NODES), (0, 0)))
    srcl, locl, nch = _filter(src, dst)
    s1, cnt = _segsum_c(xp, srcl, locl, nch)
    cnt = cnt.reshape(_NPAD, _CW)
    h = _fused(s1, cnt, xp, W1_l, W1_r, b1_l.reshape(1, _D), True)
    s2 = _segsum_n(h, srcl, locl, nch)
    out = _fused(s2, cnt, h, W2_l, W2_r, b2_l.reshape(1, _D), False)
    return out[:_NODES]


# speedup vs baseline: 5.1111x; 1.0318x over previous
import functools

import jax
import jax.numpy as jnp
from jax import lax
from jax.experimental import pallas as pl
from jax.experimental.pallas import tpu as pltpu
from jax.experimental.pallas import tpu_sc as plsc

_NODES = 10000
_EDGES = 160000
_D = 256

_NW = 32
_WIN = 320
_NPAD = _NW * _WIN
_TRASH = _WIN
_AROWS = _WIN + 8
_CW = 16

_CF = 8000
_NFC = _EDGES // _CF
_C = 64
_EMAX = 160768

_mesh = plsc.VectorSubcoreMesh(core_axis_name="c", subcore_axis_name="s")
_sc_params = pltpu.CompilerParams(needs_layout_passes=False)


@functools.partial(
    pl.kernel,
    out_type=(
        jax.ShapeDtypeStruct((_NW * _EMAX,), jnp.int32),
        jax.ShapeDtypeStruct((_NW * _EMAX,), jnp.int32),
        jax.ShapeDtypeStruct((_NW * _CW,), jnp.int32),
    ),
    mesh=_mesh,
    compiler_params=_sc_params,
    scratch_types=[
        pltpu.VMEM((_CF,), jnp.int32),
        pltpu.VMEM((_CF,), jnp.int32),
        pltpu.VMEM((_CF,), jnp.int32),
        pltpu.VMEM((_CF,), jnp.int32),
        pltpu.VMEM((_CF + 16,), jnp.int32),
        pltpu.VMEM((_CF + 16,), jnp.int32),
        pltpu.VMEM((_CF + 16,), jnp.int32),
        pltpu.VMEM((_CF + 16,), jnp.int32),
        pltpu.VMEM((_CW,), jnp.int32),
        pltpu.SemaphoreType.DMA,
        pltpu.SemaphoreType.DMA,
        pltpu.SemaphoreType.DMA,
        pltpu.SemaphoreType.DMA,
    ],
)
def _filter(src_hbm, dst_hbm, srcl_hbm, locl_hbm, nch_hbm,
            fsrc0, fsrc1, fdst0, fdst1, csrc0, csrc1, cloc0, cloc1, nch_v,
            semL0, semL1, semS0, semS1):
    c = lax.axis_index("c")
    s = lax.axis_index("s")
    w = c * 16 + s
    base = w * _WIN
    lbase = w * _EMAX

    padsrc16 = jnp.full((16,), 1, jnp.int32) * base
    trash16 = jnp.full((16,), _TRASH, jnp.int32)
    win_u = jnp.uint32(_WIN)

    def _lstart(i, fsrcb, fdstb, semL):
        @pl.when(i < _NFC)
        def _():
            pltpu.async_copy(src_hbm.at[pl.ds(i * _CF, _CF)], fsrcb, semL)
            pltpu.async_copy(dst_hbm.at[pl.ds(i * _CF, _CF)], fdstb, semL)

    def _lwait(i, fsrcb, fdstb, semL):
        pltpu.make_async_copy(src_hbm.at[pl.ds(i * _CF, _CF)], fsrcb,
                              semL).wait()
        pltpu.make_async_copy(dst_hbm.at[pl.ds(i * _CF, _CF)], fdstb,
                              semL).wait()

    def _swait(off, csrcb, clocb, semS):
        off = pl.multiple_of(off, 16)
        pltpu.make_async_copy(csrcb, srcl_hbm.at[pl.ds(off, _CF + 16)],
                              semS).wait()
        pltpu.make_async_copy(clocb, locl_hbm.at[pl.ds(off, _CF + 16)],
                              semS).wait()

    def _compact(fsrcb, fdstb, csrcb, clocb):
        def _vec(j, n):
            o = j * 32
            d0 = fdstb[pl.ds(o, 16)]
            s0 = fsrcb[pl.ds(o, 16)]
            d1 = fdstb[pl.ds(o + 16, 16)]
            s1 = fsrcb[pl.ds(o + 16, 16)]
            loc0 = d0 - base
            ok0 = loc0.astype(jnp.uint32) < win_u
            plsc.store_compressed(csrcb.at[pl.ds(n, 16)], s0, mask=ok0)
            plsc.store_compressed(clocb.at[pl.ds(n, 16)], loc0, mask=ok0)
            n = n + plsc.all_reduce_population_count(ok0)[0]
            loc1 = d1 - base
            ok1 = loc1.astype(jnp.uint32) < win_u
            plsc.store_compressed(csrcb.at[pl.ds(n, 16)], s1, mask=ok1)
            plsc.store_compressed(clocb.at[pl.ds(n, 16)], loc1, mask=ok1)
            return n + plsc.all_reduce_population_count(ok1)[0]

        n = lax.fori_loop(0, _CF // 32, _vec, jnp.int32(0))
        csrcb[pl.ds(n, 16)] = padsrc16
        clocb[pl.ds(n, 16)] = trash16
        return ((n + 15) // 16) * 16

    _lstart(0, fsrc0, fdst0, semL0)
    _lstart(1, fsrc1, fdst1, semL1)

    def _scan2(g, carry):
        out_off, offB = carry
        i0 = g * 2

        _lwait(i0, fsrc0, fdst0, semL0)
        n_r0 = _compact(fsrc0, fdst0, csrc0, cloc0)

        @pl.when(g > 0)
        def _():
            _swait(offB, csrc1, cloc1, semS1)

        dst0 = pl.multiple_of(lbase + out_off, 16)
        pltpu.async_copy(csrc0, srcl_hbm.at[pl.ds(dst0, _CF + 16)], semS0)
        pltpu.async_copy(cloc0, locl_hbm.at[pl.ds(dst0, _CF + 16)], semS0)
        _lstart(i0 + 2, fsrc0, fdst0, semL0)
        out_off = out_off + n_r0

        _lwait(i0 + 1, fsrc1, fdst1, semL1)
        n_r1 = _compact(fsrc1, fdst1, csrc1, cloc1)

        _swait(dst0, csrc0, cloc0, semS0)

        dst1 = pl.multiple_of(lbase + out_off, 16)
        pltpu.async_copy(csrc1, srcl_hbm.at[pl.ds(dst1, _CF + 16)], semS1)
        pltpu.async_copy(cloc1, locl_hbm.at[pl.ds(dst1, _CF + 16)], semS1)
        _lstart(i0 + 3, fsrc1, fdst1, semL1)
        return (out_off + n_r1, dst1)

    n_tot, offB = lax.fori_loop(0, _NFC // 2, _scan2, (jnp.int32(0),
                                                       jnp.int32(lbase)))

    _swait(offB, csrc1, cloc1, semS1)

    for k in range(5):
        csrc0[pl.ds(k * 16, 16)] = padsrc16
        cloc0[pl.ds(k * 16, 16)] = trash16
    tail_off = pl.multiple_of(lbase + n_tot, 16)
    pltpu.sync_copy(csrc0.at[pl.ds(0, 80)], srcl_hbm.at[pl.ds(tail_off, 80)])
    pltpu.sync_copy(cloc0.at[pl.ds(0, 80)], locl_hbm.at[pl.ds(tail_off, 80)])

    nch = (n_tot + _C - 1) // _C
    nch_v[...] = jnp.full((_CW,), 1, jnp.int32) * nch
    pltpu.sync_copy(nch_v, nch_hbm.at[pl.ds(w * _CW, _CW)])


def _make_segsum(with_counts):
    out_type = [jax.ShapeDtypeStruct((_NPAD, _D), jnp.float32)]
    scratch = [
        pltpu.VMEM((_C,), jnp.int32),
        pltpu.VMEM((_C,), jnp.int32),
        pltpu.VMEM((_C + 16,), jnp.int32),
        pltpu.VMEM((_C + 16,), jnp.int32),
        pltpu.VMEM((_C, _D), jnp.float32),
        pltpu.VMEM((_C, _D), jnp.float32),
        pltpu.VMEM((_AROWS, _D), jnp.float32),
        pltpu.VMEM((_CW,), jnp.int32),
        pltpu.SemaphoreType.DMA,
        pltpu.SemaphoreType.DMA,
        pltpu.SemaphoreType.DMA,
        pltpu.SemaphoreType.DMA,
    ]
    if with_counts:
        out_type.append(jax.ShapeDtypeStruct((_NPAD * _CW,), jnp.float32))
        scratch.append(pltpu.VMEM((_AROWS * _CW,), jnp.float32))

    @functools.partial(
        pl.kernel,
        out_type=tuple(out_type) if with_counts else out_type[0],
        mesh=_mesh,
        compiler_params=_sc_params,
        scratch_types=scratch,
    )
    def _segsum(z_hbm, srcl_hbm, locl_hbm, nch_hbm, out_hbm, *rest):
        if with_counts:
            (cnt_hbm, src0, src1, loc0, loc1, rows0, rows1, acc_v, nch_v,
             sem0, sem1, seml0, seml1, cnt_v) = rest
        else:
            (src0, src1, loc0, loc1, rows0, rows1, acc_v, nch_v,
             sem0, sem1, seml0, seml1) = rest
        c = lax.axis_index("c")
        s = lax.axis_index("s")
        w = c * 16 + s
        base = w * _WIN
        lbase = w * _EMAX

        zero16 = jnp.zeros((16,), jnp.float32)
        one16 = jnp.ones((16,), jnp.float32)

        def _zacc(r, carry):
            for k in range(_D // 16):
                acc_v[r, pl.ds(k * 16, 16)] = zero16
            if with_counts:
                cnt_v[pl.ds(r * _CW, _CW)] = zero16
            return carry

        lax.fori_loop(0, _AROWS, _zacc, 0)

        pltpu.sync_copy(nch_hbm.at[pl.ds(w * _CW, _CW)], nch_v)
        nch = nch_v[...][0]

        def _lstart(i, srcb, locb, seml):
            @pl.when(i < nch)
            def _():
                off = pl.multiple_of(lbase + i * _C, 16)
                pltpu.async_copy(srcl_hbm.at[pl.ds(off, _C)], srcb, seml)
                pltpu.async_copy(locl_hbm.at[pl.ds(off, _C)],
                                 locb.at[pl.ds(0, _C)], seml)

        def _lwait(i, srcb, locb, seml):
            off = pl.multiple_of(lbase + i * _C, 16)
            pltpu.make_async_copy(srcl_hbm.at[pl.ds(off, _C)], srcb,
                                  seml).wait()
            pltpu.make_async_copy(locl_hbm.at[pl.ds(off, _C)],
                                  locb.at[pl.ds(0, _C)], seml).wait()

        def _finish(srcb, rowsb, sem):
            pltpu.make_async_copy(z_hbm.at[srcb], rowsb, sem).wait()

        def _edges(rowsb, locb):
            def _grp(j8, carry):
                jb = j8 * 8
                locs = locb[pl.ds(jb, 16)]
                ls = [locs[l] for l in range(8)]
                if with_counts:
                    for l in range(8):
                        plsc.addupdate(cnt_v.at[pl.ds(ls[l] * _CW, _CW)],
                                       one16)
                for l in range(8):
                    row = jb + l
                    vals = [rowsb[row, pl.ds(k * 16, 16)]
                            for k in range(_D // 16)]
                    for k in range(_D // 16):
                        plsc.addupdate(acc_v.at[ls[l], pl.ds(k * 16, 16)],
                                       vals[k])
                return carry

            lax.fori_loop(0, _C // 8, _grp, 0)

        @pl.when(0 < nch)
        def _():
            off = pl.multiple_of(lbase, 16)
            pltpu.sync_copy(srcl_hbm.at[pl.ds(off, _C)], src0)
            pltpu.sync_copy(locl_hbm.at[pl.ds(off, _C)], loc0.at[pl.ds(0, _C)])
            pltpu.async_copy(z_hbm.at[src0], rows0, sem0)

        _lstart(1, src1, loc1, seml1)

        def _outer(g, carry):
            i0 = g * 2
            _finish(src0, rows0, sem0)

            @pl.when(i0 + 1 < nch)
            def _():
                _lwait(i0 + 1, src1, loc1, seml1)
                pltpu.async_copy(z_hbm.at[src1], rows1, sem1)

            _edges(rows0, loc0)
            _lstart(i0 + 2, src0, loc0, seml0)

            @pl.when(i0 + 1 < nch)
            def _():
                _finish(src1, rows1, sem1)

                @pl.when(i0 + 2 < nch)
                def _():
                    _lwait(i0 + 2, src0, loc0, seml0)
                    pltpu.async_copy(z_hbm.at[src0], rows0, sem0)

                _edges(rows1, loc1)
                _lstart(i0 + 3, src1, loc1, seml1)

            return carry

        lax.fori_loop(0, (nch + 1) // 2, _outer, 0)

        pltpu.sync_copy(acc_v.at[pl.ds(0, _WIN)], out_hbm.at[pl.ds(base, _WIN)])
        if with_counts:
            pltpu.sync_copy(cnt_v.at[pl.ds(0, _WIN * _CW)],
                            cnt_hbm.at[pl.ds(base * _CW, _WIN * _CW)])

    return _segsum


_segsum_c = _make_segsum(True)
_segsum_n = _make_segsum(False)


_BR = 1000


def _fused_body(s_ref, c_ref, x_ref, wl_ref, wr_ref, b_ref, o_ref, *, elu):
    cnt = jnp.maximum(c_ref[...][:, 0:1], 1.0)
    mean = s_ref[...] / cnt
    acc = lax.dot_general(mean, wl_ref[...], (((1,), (1,)), ((), ())),
                          precision=lax.Precision.HIGHEST,
                          preferred_element_type=jnp.float32)
    acc = acc + lax.dot_general(x_ref[...], wr_ref[...], (((1,), (1,)), ((), ())),
                                precision=lax.Precision.HIGHEST,
                                preferred_element_type=jnp.float32)
    acc = acc + b_ref[...]
    if elu:
        acc = jnp.where(acc > 0.0, acc, jnp.exp(jnp.minimum(acc, 0.0)) - 1.0)
    o_ref[...] = acc


def _fused(ssum, cnt, x, w_l, w_r, b, elu):
    return pl.pallas_call(
        functools.partial(_fused_body, elu=elu),
        grid=(_NODES // _BR,),
        in_specs=[
            pl.BlockSpec((_BR, _D), lambda i: (i, 0)),
            pl.BlockSpec((_BR, _CW), lambda i: (i, 0)),
            pl.BlockSpec((_BR, _D), lambda i: (i, 0)),
            pl.BlockSpec((_D, _D), lambda i: (0, 0)),
            pl.BlockSpec((_D, _D), lambda i: (0, 0)),
            pl.BlockSpec((1, _D), lambda i: (0, 0)),
        ],
        out_specs=pl.BlockSpec((_BR, _D), lambda i: (i, 0)),
        out_shape=jax.ShapeDtypeStruct((_NODES, _D), jnp.float32),
    )(ssum, cnt, x, w_l, w_r, b)


def kernel(x, edge_index, X_param, W1_l, b1_l, W1_r, W2_l, b2_l, W2_r):
    del x
    src = edge_index[0].astype(jnp.int32)
    dst = edge_index[1].astype(jnp.int32)
    srcl, locl, nch = _filter(src, dst)
    s1, cnt = _segsum_c(X_param, srcl, locl, nch)
    cnt = cnt.reshape(_NPAD, _CW)
    h = _fused(s1, cnt, X_param, W1_l, W1_r, b1_l.reshape(1, _D), True)
    s2 = _segsum_n(h, srcl, locl, nch)
    return _fused(s2, cnt, h, W2_l, W2_r, b2_l.reshape(1, _D), False)
